# initial kernel scaffold (unmeasured)
import jax
import jax.numpy as jnp
from jax import lax
from jax.experimental import pallas as pl
from jax.experimental.pallas import tpu as pltpu

N_DEV = 16


def kernel(x, W1, W2):
    m, k = x.shape
    h = W1.shape[1]
    n = W2.shape[1]

    def body(x_ref, w1_ref, w2_ref, out_ref,
             send_buf, recv_buf, send_sems, recv_sems):
        my = lax.axis_index("i")

        xb = x_ref[...].astype(jnp.bfloat16)
        w1b = w1_ref[...].astype(jnp.bfloat16)
        w2b = w2_ref[...].astype(jnp.bfloat16)
        hh = jnp.maximum(jnp.dot(xb, w1b, preferred_element_type=jnp.float32), 0.0)
        partial = jnp.dot(hh.astype(jnp.bfloat16), w2b,
                          preferred_element_type=jnp.float32)
        send_buf[...] = partial.astype(jnp.bfloat16)

        sends = []
        for kk in range(1, N_DEV):
            tgt = lax.rem(my + kk, N_DEV)
            rdma = pltpu.make_async_remote_copy(
                src_ref=send_buf,
                dst_ref=recv_buf.at[kk - 1],
                send_sem=send_sems.at[kk - 1],
                recv_sem=recv_sems.at[kk - 1],
                device_id=(tgt,),
                device_id_type=pl.DeviceIdType.MESH,
            )
            rdma.start()
            sends.append(rdma)

        for kk in range(1, N_DEV):
            recv = pltpu.make_async_remote_copy(
                src_ref=send_buf,
                dst_ref=recv_buf.at[kk - 1],
                send_sem=send_sems.at[kk - 1],
                recv_sem=recv_sems.at[kk - 1],
                device_id=(my,),
                device_id_type=pl.DeviceIdType.MESH,
            )
            recv.wait_recv()

        for rdma in sends:
            rdma.wait_send()

        out_ref[...] = partial + jnp.sum(
            recv_buf[...].astype(jnp.float32), axis=0
        )

    return pl.pallas_call(
        body,
        out_shape=jax.ShapeDtypeStruct((m, n), jnp.float32),
        in_specs=[
            pl.BlockSpec(memory_space=pltpu.VMEM),
            pl.BlockSpec(memory_space=pltpu.VMEM),
            pl.BlockSpec(memory_space=pltpu.VMEM),
        ],
        out_specs=pl.BlockSpec(memory_space=pltpu.VMEM),
        scratch_shapes=[
            pltpu.VMEM((m, n), jnp.bfloat16),
            pltpu.VMEM((N_DEV - 1, m, n), jnp.bfloat16),
            pltpu.SemaphoreType.DMA((N_DEV - 1,)),
            pltpu.SemaphoreType.DMA((N_DEV - 1,)),
        ],
        compiler_params=pltpu.CompilerParams(collective_id=0),
    )(x, W1, W2)


# baseline (device time: 34360 ns/iter reference)
import jax
import jax.numpy as jnp
from jax import lax
from jax.experimental import pallas as pl
from jax.experimental.pallas import tpu as pltpu

N_DEV = 16


def kernel(x, W1, W2):
    m, k = x.shape
    h = W1.shape[1]
    n = W2.shape[1]

    def body(x_ref, w1_ref, w2_ref, out_ref,
             send_buf, recv_buf, send_sems, recv_sems):
        my = lax.axis_index("i")

        xb = x_ref[...].astype(jnp.bfloat16)
        w1b = w1_ref[...].astype(jnp.bfloat16)
        w2b = w2_ref[...].astype(jnp.bfloat16)
        hh = jnp.maximum(jnp.dot(xb, w1b, preferred_element_type=jnp.float32), 0.0)
        partial = jnp.dot(hh.astype(jnp.bfloat16), w2b,
                          preferred_element_type=jnp.float32)
        send_buf[...] = partial.astype(jnp.bfloat16)

        sends = []
        for kk in range(1, N_DEV):
            tgt = lax.rem(my + kk, N_DEV)
            rdma = pltpu.make_async_remote_copy(
                src_ref=send_buf,
                dst_ref=recv_buf.at[kk - 1],
                send_sem=send_sems.at[kk - 1],
                recv_sem=recv_sems.at[kk - 1],
                device_id=(tgt,),
                device_id_type=pl.DeviceIdType.MESH,
            )
            rdma.start()
            sends.append(rdma)

        for kk in range(1, N_DEV):
            recv = pltpu.make_async_remote_copy(
                src_ref=send_buf,
                dst_ref=recv_buf.at[kk - 1],
                send_sem=send_sems.at[kk - 1],
                recv_sem=recv_sems.at[kk - 1],
                device_id=(my,),
                device_id_type=pl.DeviceIdType.MESH,
            )
            recv.wait_recv()

        for rdma in sends:
            rdma.wait_send()

        out_ref[...] = partial + jnp.sum(
            recv_buf[...].astype(jnp.float32), axis=0
        )

    return pl.pallas_call(
        body,
        out_shape=jax.ShapeDtypeStruct((m, n), jnp.float32),
        in_specs=[
            pl.BlockSpec(memory_space=pltpu.VMEM),
            pl.BlockSpec(memory_space=pltpu.VMEM),
            pl.BlockSpec(memory_space=pltpu.VMEM),
        ],
        out_specs=pl.BlockSpec(memory_space=pltpu.VMEM),
        scratch_shapes=[
            pltpu.VMEM((m, n), jnp.bfloat16),
            pltpu.VMEM((N_DEV - 1, m, n), jnp.bfloat16),
            pltpu.SemaphoreType.DMA((N_DEV - 1,)),
            pltpu.SemaphoreType.DMA((N_DEV - 1,)),
        ],
    )(x, W1, W2)


# device time: 22085 ns/iter; 1.5558x vs baseline; 1.5558x over previous
import jax
import jax.numpy as jnp
from jax import lax
from jax.experimental import pallas as pl
from jax.experimental.pallas import tpu as pltpu

N_DEV = 16


def kernel(x, W1, W2):
    m, k = x.shape
    n = W2.shape[1]
    rows = m // N_DEV

    def body(x_ref, w1_ref, w2_ref, out_ref,
             send_buf, rs_buf, red_buf,
             send_sems1, recv_sems1, send_sems2, recv_sems2):
        my = lax.axis_index("i")

        xb = x_ref[...].astype(jnp.bfloat16)
        w1b = w1_ref[...].astype(jnp.bfloat16)
        w2b = w2_ref[...].astype(jnp.bfloat16)
        hh = jnp.maximum(jnp.dot(xb, w1b, preferred_element_type=jnp.float32), 0.0)
        partial = jnp.dot(hh.astype(jnp.bfloat16), w2b,
                          preferred_element_type=jnp.float32)
        send_buf[...] = partial.astype(jnp.bfloat16)

        sends1 = []
        for kk in range(1, N_DEV):
            tgt = lax.rem(my + kk, N_DEV)
            rdma = pltpu.make_async_remote_copy(
                src_ref=send_buf.at[pl.ds(tgt * rows, rows)],
                dst_ref=rs_buf.at[kk - 1],
                send_sem=send_sems1.at[kk - 1],
                recv_sem=recv_sems1.at[kk - 1],
                device_id=(tgt,),
                device_id_type=pl.DeviceIdType.MESH,
            )
            rdma.start()
            sends1.append(rdma)

        for kk in range(1, N_DEV):
            recv = pltpu.make_async_remote_copy(
                src_ref=send_buf.at[pl.ds(0, rows)],
                dst_ref=rs_buf.at[kk - 1],
                send_sem=send_sems1.at[kk - 1],
                recv_sem=recv_sems1.at[kk - 1],
                device_id=(my,),
                device_id_type=pl.DeviceIdType.MESH,
            )
            recv.wait_recv()

        own = send_buf[pl.ds(my * rows, rows), :].astype(jnp.float32)
        red = own + jnp.sum(rs_buf[...].astype(jnp.float32), axis=0)
        red_buf[...] = red
        out_ref[pl.ds(my * rows, rows), :] = red

        sends2 = []
        for kk in range(1, N_DEV):
            tgt = lax.rem(my + kk, N_DEV)
            rdma = pltpu.make_async_remote_copy(
                src_ref=red_buf,
                dst_ref=out_ref.at[pl.ds(my * rows, rows)],
                send_sem=send_sems2.at[kk - 1],
                recv_sem=recv_sems2.at[kk - 1],
                device_id=(tgt,),
                device_id_type=pl.DeviceIdType.MESH,
            )
            rdma.start()
            sends2.append(rdma)

        for kk in range(1, N_DEV):
            recv = pltpu.make_async_remote_copy(
                src_ref=red_buf,
                dst_ref=out_ref.at[pl.ds(0, rows)],
                send_sem=send_sems2.at[kk - 1],
                recv_sem=recv_sems2.at[kk - 1],
                device_id=(my,),
                device_id_type=pl.DeviceIdType.MESH,
            )
            recv.wait_recv()

        for rdma in sends1:
            rdma.wait_send()
        for rdma in sends2:
            rdma.wait_send()

    return pl.pallas_call(
        body,
        out_shape=jax.ShapeDtypeStruct((m, n), jnp.float32),
        in_specs=[
            pl.BlockSpec(memory_space=pltpu.VMEM),
            pl.BlockSpec(memory_space=pltpu.VMEM),
            pl.BlockSpec(memory_space=pltpu.VMEM),
        ],
        out_specs=pl.BlockSpec(memory_space=pltpu.VMEM),
        scratch_shapes=[
            pltpu.VMEM((m, n), jnp.bfloat16),
            pltpu.VMEM((N_DEV - 1, rows, n), jnp.bfloat16),
            pltpu.VMEM((rows, n), jnp.float32),
            pltpu.SemaphoreType.DMA((N_DEV - 1,)),
            pltpu.SemaphoreType.DMA((N_DEV - 1,)),
            pltpu.SemaphoreType.DMA((N_DEV - 1,)),
            pltpu.SemaphoreType.DMA((N_DEV - 1,)),
        ],
    )(x, W1, W2)


# device time: 15857 ns/iter; 2.1669x vs baseline; 1.3928x over previous
import jax
import jax.numpy as jnp
from jax import lax
from jax.experimental import pallas as pl
from jax.experimental.pallas import tpu as pltpu

N_DEV = 16


def kernel(x, W1, W2):
    m, k = x.shape
    n = W2.shape[1]
    rows = m // N_DEV

    def body(x_ref, w1_ref, w2_ref, out_ref,
             send_buf, rs_buf, red_buf,
             send_sems1, recv_sems1, send_sems2, recv_sems2):
        my = lax.axis_index("i")

        barrier_sem = pltpu.get_barrier_semaphore()
        for kk in range(1, N_DEV):
            pl.semaphore_signal(
                barrier_sem, inc=1,
                device_id=(lax.rem(my + kk, N_DEV),),
                device_id_type=pl.DeviceIdType.MESH,
            )

        xb = x_ref[...].astype(jnp.bfloat16)
        w1b = w1_ref[...].astype(jnp.bfloat16)
        w2b = w2_ref[...].astype(jnp.bfloat16)
        hh = jnp.maximum(jnp.dot(xb, w1b, preferred_element_type=jnp.float32), 0.0)
        partial = jnp.dot(hh.astype(jnp.bfloat16), w2b,
                          preferred_element_type=jnp.float32)
        send_buf[...] = partial.astype(jnp.bfloat16)

        pl.semaphore_wait(barrier_sem, N_DEV - 1)

        sends1 = []
        for kk in range(1, N_DEV):
            tgt = lax.rem(my + kk, N_DEV)
            rdma = pltpu.make_async_remote_copy(
                src_ref=send_buf.at[pl.ds(tgt * rows, rows)],
                dst_ref=rs_buf.at[kk - 1],
                send_sem=send_sems1.at[kk - 1],
                recv_sem=recv_sems1.at[kk - 1],
                device_id=(tgt,),
                device_id_type=pl.DeviceIdType.MESH,
            )
            rdma.start()
            sends1.append(rdma)

        for kk in range(1, N_DEV):
            recv = pltpu.make_async_remote_copy(
                src_ref=send_buf.at[pl.ds(0, rows)],
                dst_ref=rs_buf.at[kk - 1],
                send_sem=send_sems1.at[kk - 1],
                recv_sem=recv_sems1.at[kk - 1],
                device_id=(my,),
                device_id_type=pl.DeviceIdType.MESH,
            )
            recv.wait_recv()

        own = send_buf[pl.ds(my * rows, rows), :].astype(jnp.float32)
        red = own + jnp.sum(rs_buf[...].astype(jnp.float32), axis=0)
        red_buf[...] = red
        out_ref[pl.ds(my * rows, rows), :] = red

        sends2 = []
        for kk in range(1, N_DEV):
            tgt = lax.rem(my + kk, N_DEV)
            rdma = pltpu.make_async_remote_copy(
                src_ref=red_buf,
                dst_ref=out_ref.at[pl.ds(my * rows, rows)],
                send_sem=send_sems2.at[kk - 1],
                recv_sem=recv_sems2.at[kk - 1],
                device_id=(tgt,),
                device_id_type=pl.DeviceIdType.MESH,
            )
            rdma.start()
            sends2.append(rdma)

        for kk in range(1, N_DEV):
            recv = pltpu.make_async_remote_copy(
                src_ref=red_buf,
                dst_ref=out_ref.at[pl.ds(0, rows)],
                send_sem=send_sems2.at[kk - 1],
                recv_sem=recv_sems2.at[kk - 1],
                device_id=(my,),
                device_id_type=pl.DeviceIdType.MESH,
            )
            recv.wait_recv()

        for rdma in sends1:
            rdma.wait_send()
        for rdma in sends2:
            rdma.wait_send()

    return pl.pallas_call(
        body,
        out_shape=jax.ShapeDtypeStruct((m, n), jnp.float32),
        in_specs=[
            pl.BlockSpec(memory_space=pltpu.VMEM),
            pl.BlockSpec(memory_space=pltpu.VMEM),
            pl.BlockSpec(memory_space=pltpu.VMEM),
        ],
        out_specs=pl.BlockSpec(memory_space=pltpu.VMEM),
        scratch_shapes=[
            pltpu.VMEM((m, n), jnp.bfloat16),
            pltpu.VMEM((N_DEV - 1, rows, n), jnp.bfloat16),
            pltpu.VMEM((rows, n), jnp.float32),
            pltpu.SemaphoreType.DMA((N_DEV - 1,)),
            pltpu.SemaphoreType.DMA((N_DEV - 1,)),
            pltpu.SemaphoreType.DMA((N_DEV - 1,)),
            pltpu.SemaphoreType.DMA((N_DEV - 1,)),
        ],
        compiler_params=pltpu.CompilerParams(collective_id=0),
    )(x, W1, W2)
